# baseline (speedup 1.0000x reference)
import functools
import math

import jax
import jax.numpy as jnp
from jax import lax
from jax.experimental import pallas as pl
from jax.experimental.pallas import tpu as pltpu
from jax.experimental.pallas import tpu_sc as plsc

D = 64
SCALE = math.sqrt(D)
NC, NS = 2, 16
NW = NC * NS
IB = 128


@functools.lru_cache(maxsize=None)
def _emb_kernel(n_b1, n_b0h):
    ngroups = n_b1 * n_b0h
    g_per_w = ngroups // NW
    assert g_per_w % 2 == 0
    npairs = g_per_w // 2
    mesh = plsc.VectorSubcoreMesh(
        core_axis_name="c", subcore_axis_name="s",
        num_cores=NC, num_subcores=NS)

    @functools.partial(
        pl.kernel,
        out_type=jax.ShapeDtypeStruct((n_b1, D // 8, n_b0h, 8 * IB),
                                      jnp.float32),
        mesh=mesh,
        scratch_types=[
            pltpu.VMEM((g_per_w, IB), jnp.int32),
            pltpu.VMEM((IB, D), jnp.float32),
            pltpu.VMEM((IB, D), jnp.float32),
            pltpu.VMEM((D * IB,), jnp.float32),
            pltpu.VMEM((D * IB,), jnp.float32),
            pltpu.SemaphoreType.DMA,
            pltpu.SemaphoreType.DMA,
            pltpu.SemaphoreType.DMA,
            pltpu.SemaphoreType.DMA,
        ],
        compiler_params=pltpu.CompilerParams(use_tc_tiling_on_sc=False,
                                             needs_layout_passes=False),
    )
    def body(xcols_hbm, lut_hbm, out_hbm, idx_v, rows_a, rows_b, tp_a, tp_b,
             gsem_a, gsem_b, ssem_a, ssem_b):
        wid = lax.axis_index("s") * NC + lax.axis_index("c")
        gbase = wid * g_per_w
        pltpu.sync_copy(xcols_hbm.at[pl.ds(gbase, g_per_w)], idx_v)

        def start_gather(j, rows_v, gsem):
            pltpu.async_copy(lut_hbm.at[idx_v.at[j]], rows_v, gsem)

        def drain_gather(rows_v, gsem):
            pltpu.make_async_copy(lut_hbm.at[idx_v.at[0]], rows_v, gsem).wait()

        def transpose_scale(rows_v, tp_v):
            bases = [
                lax.iota(jnp.int32, 16) * IB + (k * 16) * IB
                for k in range(D // 16)
            ]

            def row(l, c):
                for k in range(D // 16):
                    v = rows_v[l, pl.ds(k * 16, 16)] * SCALE
                    plsc.store_scatter(tp_v, [bases[k] + l], v)
                return c

            lax.fori_loop(0, IB, row, 0)

        def start_scatter(j, tp_v, ssem):
            g = gbase + j
            b1 = g // n_b0h
            b0h = g % n_b0h
            for c8 in range(D // 8):
                pltpu.async_copy(
                    tp_v.at[pl.ds(c8 * (8 * IB), 8 * IB)],
                    out_hbm.at[b1, c8, b0h], ssem)

        def drain_scatter(j, tp_v, ssem):
            g = gbase + j
            b1 = g // n_b0h
            b0h = g % n_b0h
            for c8 in range(D // 8):
                pltpu.make_async_copy(
                    tp_v.at[pl.ds(c8 * (8 * IB), 8 * IB)],
                    out_hbm.at[b1, c8, b0h], ssem).wait()

        start_gather(0, rows_a, gsem_a)
        start_gather(1, rows_b, gsem_b)

        def pair(i2, c):
            ja = 2 * i2
            drain_gather(rows_a, gsem_a)
            transpose_scale(rows_a, tp_a)
            start_scatter(ja, tp_a, ssem_a)
            drain_gather(rows_b, gsem_b)
            transpose_scale(rows_b, tp_b)
            start_scatter(ja + 1, tp_b, ssem_b)
            drain_scatter(ja, tp_a, ssem_a)
            start_gather(ja + 2, rows_a, gsem_a)
            drain_scatter(ja + 1, tp_b, ssem_b)
            start_gather(ja + 3, rows_b, gsem_b)
            return c

        lax.fori_loop(0, npairs - 1, pair, 0)

        jl = g_per_w - 2
        drain_gather(rows_a, gsem_a)
        transpose_scale(rows_a, tp_a)
        start_scatter(jl, tp_a, ssem_a)
        drain_gather(rows_b, gsem_b)
        transpose_scale(rows_b, tp_b)
        start_scatter(jl + 1, tp_b, ssem_b)
        drain_scatter(jl, tp_a, ssem_a)
        drain_scatter(jl + 1, tp_b, ssem_b)

    return body


@jax.jit
def kernel(x, lut):
    s0, s1 = x.shape
    n_b0h = s0 // IB
    xcols = jnp.swapaxes(x, 0, 1).reshape(s1 * n_b0h, IB).astype(jnp.int32)
    out5 = _emb_kernel(s1, n_b0h)(xcols, lut)
    out = out5.reshape(s1, D // 8, n_b0h, 8, IB)
    out = out.transpose(2, 4, 0, 1, 3).reshape(s0, s1, D)
    return out
